# baseline (speedup 1.0000x reference)
import functools

import jax
import jax.numpy as jnp
from jax import lax
from jax.experimental import pallas as pl
from jax.experimental.pallas import tpu as pltpu
from jax.experimental.pallas import tpu_sc as plsc

N = 10000
E = 320000
D = 128

NC = 2
NS = 16
NW = NC * NS
CPT = D // NW
CHUNK = 8000
NCHUNK = E // CHUNK
GROUPS = CHUNK // 16
UNROLL = 20

NEG = float("-inf")


def _sc_body(xt_hbm, norm_hbm, pk_hbm, out_hbm, refs):
    x_cs = refs[0:CPT]
    acc_cs = refs[CPT:2 * CPT]
    norm_v, ebuf, sem = refs[2 * CPT:]
    wid = lax.axis_index("s") * NC + lax.axis_index("c")

    for c in range(CPT):
        pltpu.sync_copy(xt_hbm.at[wid, c], x_cs[c])
    pltpu.sync_copy(norm_hbm, norm_v)

    def init_body(j, _):
        ds = pl.ds(j * 16, 16)
        nv = norm_v[ds]
        ninf = jnp.full((16,), NEG, jnp.float32)
        for c in range(CPT):
            x_cs[c][ds] = x_cs[c][ds] * nv
            acc_cs[c][ds] = ninf
        return _

    lax.fori_loop(0, N // 16, init_body, None)

    def group_front(base):
        p = ebuf[pl.ds(base, 16)]
        src = p & 16383
        dst = lax.shift_right_logical(p, 14)
        vks, dks = [], []
        for c in range(CPT):
            xv = plsc.load_gather(x_cs[c], [src])
            vk, dk = plsc.sort_key_val(xv, dst)
            vks.append(vk)
            dks.append(dk)
        return vks, dks

    def group_back(vks, dks):
        olds = [plsc.load_gather(acc_cs[c], [dks[c]]) for c in range(CPT)]
        for c in range(CPT):
            plsc.store_scatter(acc_cs[c], [dks[c]], jnp.maximum(olds[c], vks[c]))

    def chunk_body(ci, _):
        slot = (ci & 1) * CHUNK
        pltpu.make_async_copy(
            pk_hbm.at[pl.ds(ci * CHUNK, CHUNK)],
            ebuf.at[pl.ds(slot, CHUNK)],
            sem,
        ).wait()

        nxt = ci + 1

        @pl.when(nxt < NCHUNK)
        def _start_next():
            pltpu.async_copy(
                pk_hbm.at[pl.ds(nxt * CHUNK, CHUNK)],
                ebuf.at[pl.ds((nxt & 1) * CHUNK, CHUNK)],
                sem,
            )

        def group_body(g, _):
            fronts = [
                group_front(slot + (g * UNROLL + u) * 16) for u in range(UNROLL)
            ]
            for vks, dks in fronts:
                group_back(vks, dks)
            return _

        lax.fori_loop(0, GROUPS // UNROLL, group_body, None)
        return _

    pltpu.async_copy(pk_hbm.at[pl.ds(0, CHUNK)], ebuf.at[pl.ds(0, CHUNK)], sem)
    lax.fori_loop(0, NCHUNK, chunk_body, None)

    def out_body(j, _):
        ds = pl.ds(j * 16, 16)
        nv = norm_v[ds]
        for c in range(CPT):
            v = acc_cs[c][ds]
            acc_cs[c][ds] = jnp.where(v == NEG, jnp.float32(0.0), v * nv)
        return _

    lax.fori_loop(0, N // 16, out_body, None)
    for c in range(CPT):
        pltpu.sync_copy(acc_cs[c], out_hbm.at[wid, c])


@functools.partial(
    pl.kernel,
    out_type=jax.ShapeDtypeStruct((NW, CPT, N), jnp.float32),
    mesh=plsc.VectorSubcoreMesh(core_axis_name="c", subcore_axis_name="s"),
    compiler_params=pltpu.CompilerParams(needs_layout_passes=False),
    scratch_types=(
        [pltpu.VMEM((N,), jnp.float32) for _ in range(2 * CPT)]
        + [
            pltpu.VMEM((N,), jnp.float32),
            pltpu.VMEM((2 * CHUNK,), jnp.int32),
            pltpu.SemaphoreType.DMA,
        ]
    ),
)
def _sc_kernel(xt_hbm, norm_hbm, pk_hbm, out_hbm, *refs):
    _sc_body(xt_hbm, norm_hbm, pk_hbm, out_hbm, refs)


def kernel(x, norm, edge_index):
    xt = x.reshape(N, NW, CPT).transpose(1, 2, 0)
    ei = edge_index.astype(jnp.int32)
    packed = (ei[1] << 14) | ei[0]
    out = _sc_kernel(xt, norm.reshape(N), packed)
    return out.transpose(2, 0, 1).reshape(N, D)


# speedup vs baseline: 1.0650x; 1.0650x over previous
import functools

import jax
import jax.numpy as jnp
from jax import lax
from jax.experimental import pallas as pl
from jax.experimental.pallas import tpu as pltpu
from jax.experimental.pallas import tpu_sc as plsc

N = 10000
E = 320000
D = 128

NC = 2
NS = 16
NW = NC * NS
CPT = D // NW
CHUNK = 8000
NCHUNK = E // CHUNK
GROUPS = CHUNK // 16
UNROLL = 8

NEG = float("-inf")


def _sc_body(xt_hbm, norm_hbm, pk_hbm, out_hbm, refs):
    x_cs = refs[0:CPT]
    acc_cs = refs[CPT:2 * CPT]
    norm_v, ebuf, sem = refs[2 * CPT:]
    wid = lax.axis_index("s") * NC + lax.axis_index("c")

    for c in range(CPT):
        pltpu.sync_copy(xt_hbm.at[wid, c], x_cs[c])
    pltpu.sync_copy(norm_hbm, norm_v)

    def init_body(j, _):
        ds = pl.ds(j * 16, 16)
        nv = norm_v[ds]
        ninf = jnp.full((16,), NEG, jnp.float32)
        for c in range(CPT):
            x_cs[c][ds] = x_cs[c][ds] * nv
            acc_cs[c][ds] = ninf
        return _

    lax.fori_loop(0, N // 16, init_body, None)

    def group_front(base):
        p = ebuf[pl.ds(base, 16)]
        src = p & 16383
        dst = lax.shift_right_logical(p, 14)
        vks, dks = [], []
        for c in range(CPT):
            xv = plsc.load_gather(x_cs[c], [src])
            vk, dk = plsc.sort_key_val(xv, dst)
            vks.append(vk)
            dks.append(dk)
        return vks, dks

    def group_back(vks, dks):
        olds = [plsc.load_gather(acc_cs[c], [dks[c]]) for c in range(CPT)]
        for c in range(CPT):
            plsc.store_scatter(acc_cs[c], [dks[c]], jnp.maximum(olds[c], vks[c]))

    def chunk_body(ci, _):
        slot = (ci & 1) * CHUNK
        pltpu.make_async_copy(
            pk_hbm.at[pl.ds(ci * CHUNK, CHUNK)],
            ebuf.at[pl.ds(slot, CHUNK)],
            sem,
        ).wait()

        nxt = ci + 1

        @pl.when(nxt < NCHUNK)
        def _start_next():
            pltpu.async_copy(
                pk_hbm.at[pl.ds(nxt * CHUNK, CHUNK)],
                ebuf.at[pl.ds((nxt & 1) * CHUNK, CHUNK)],
                sem,
            )

        def group_body(g, _):
            fronts = [
                group_front(slot + (g * UNROLL + u) * 16) for u in range(UNROLL)
            ]
            for vks, dks in fronts:
                group_back(vks, dks)
            return _

        lax.fori_loop(0, GROUPS // UNROLL, group_body, None)
        return _

    pltpu.async_copy(pk_hbm.at[pl.ds(0, CHUNK)], ebuf.at[pl.ds(0, CHUNK)], sem)
    lax.fori_loop(0, NCHUNK, chunk_body, None)

    def out_body(j, _):
        ds = pl.ds(j * 16, 16)
        nv = norm_v[ds]
        for c in range(CPT):
            v = acc_cs[c][ds]
            acc_cs[c][ds] = jnp.where(v == NEG, jnp.float32(0.0), v * nv)
        return _

    lax.fori_loop(0, N // 16, out_body, None)
    for c in range(CPT):
        pltpu.sync_copy(acc_cs[c], out_hbm.at[wid, c])


@functools.partial(
    pl.kernel,
    out_type=jax.ShapeDtypeStruct((NW, CPT, N), jnp.float32),
    mesh=plsc.VectorSubcoreMesh(core_axis_name="c", subcore_axis_name="s"),
    compiler_params=pltpu.CompilerParams(needs_layout_passes=False),
    scratch_types=(
        [pltpu.VMEM((N,), jnp.float32) for _ in range(2 * CPT)]
        + [
            pltpu.VMEM((N,), jnp.float32),
            pltpu.VMEM((2 * CHUNK,), jnp.int32),
            pltpu.SemaphoreType.DMA,
        ]
    ),
)
def _sc_kernel(xt_hbm, norm_hbm, pk_hbm, out_hbm, *refs):
    _sc_body(xt_hbm, norm_hbm, pk_hbm, out_hbm, refs)


def kernel(x, norm, edge_index):
    xt = x.reshape(N, NW, CPT).transpose(1, 2, 0)
    ei = edge_index.astype(jnp.int32)
    packed = (ei[1] << 14) | ei[0]
    out = _sc_kernel(xt, norm.reshape(N), packed)
    return out.transpose(2, 0, 1).reshape(N, D)


# speedup vs baseline: 1.0720x; 1.0066x over previous
import functools

import jax
import jax.numpy as jnp
from jax import lax
from jax.experimental import pallas as pl
from jax.experimental.pallas import tpu as pltpu
from jax.experimental.pallas import tpu_sc as plsc

N = 10000
E = 320000
D = 128

NC = 2
NS = 16
NW = NC * NS
CPT = D // NW
CHUNK = 16000
NCHUNK = E // CHUNK
GROUPS = CHUNK // 16
UNROLL = 10

NEG = float("-inf")


def _sc_body(xt_hbm, norm_hbm, pk_hbm, out_hbm, refs):
    x_cs = refs[0:CPT]
    acc_cs = refs[CPT:2 * CPT]
    norm_v, ebuf, sem = refs[2 * CPT:]
    wid = lax.axis_index("s") * NC + lax.axis_index("c")

    for c in range(CPT):
        pltpu.sync_copy(xt_hbm.at[wid, c], x_cs[c])
    pltpu.sync_copy(norm_hbm, norm_v)

    def init_body(j, _):
        ds = pl.ds(j * 16, 16)
        nv = norm_v[ds]
        ninf = jnp.full((16,), NEG, jnp.float32)
        for c in range(CPT):
            x_cs[c][ds] = x_cs[c][ds] * nv
            acc_cs[c][ds] = ninf
        return _

    lax.fori_loop(0, N // 16, init_body, None)

    def group_front(base):
        p = ebuf[pl.ds(base, 16)]
        src = p & 16383
        dst = lax.shift_right_logical(p, 14)
        vks, dks = [], []
        for c in range(CPT):
            xv = plsc.load_gather(x_cs[c], [src])
            vk, dk = plsc.sort_key_val(xv, dst)
            vks.append(vk)
            dks.append(dk)
        return vks, dks

    def group_back(vks, dks):
        olds = [plsc.load_gather(acc_cs[c], [dks[c]]) for c in range(CPT)]
        for c in range(CPT):
            plsc.store_scatter(acc_cs[c], [dks[c]], jnp.maximum(olds[c], vks[c]))

    def chunk_body(ci, _):
        slot = (ci & 1) * CHUNK
        pltpu.make_async_copy(
            pk_hbm.at[pl.ds(ci * CHUNK, CHUNK)],
            ebuf.at[pl.ds(slot, CHUNK)],
            sem,
        ).wait()

        nxt = ci + 1

        @pl.when(nxt < NCHUNK)
        def _start_next():
            pltpu.async_copy(
                pk_hbm.at[pl.ds(nxt * CHUNK, CHUNK)],
                ebuf.at[pl.ds((nxt & 1) * CHUNK, CHUNK)],
                sem,
            )

        def group_body(g, _):
            fronts = [
                group_front(slot + (g * UNROLL + u) * 16) for u in range(UNROLL)
            ]
            for vks, dks in fronts:
                group_back(vks, dks)
            return _

        lax.fori_loop(0, GROUPS // UNROLL, group_body, None)
        return _

    pltpu.async_copy(pk_hbm.at[pl.ds(0, CHUNK)], ebuf.at[pl.ds(0, CHUNK)], sem)
    lax.fori_loop(0, NCHUNK, chunk_body, None)

    def out_body(j, _):
        ds = pl.ds(j * 16, 16)
        nv = norm_v[ds]
        for c in range(CPT):
            v = acc_cs[c][ds]
            acc_cs[c][ds] = jnp.where(v == NEG, jnp.float32(0.0), v * nv)
        return _

    lax.fori_loop(0, N // 16, out_body, None)
    for c in range(CPT):
        pltpu.sync_copy(acc_cs[c], out_hbm.at[wid, c])


@functools.partial(
    pl.kernel,
    out_type=jax.ShapeDtypeStruct((NW, CPT, N), jnp.float32),
    mesh=plsc.VectorSubcoreMesh(core_axis_name="c", subcore_axis_name="s"),
    compiler_params=pltpu.CompilerParams(needs_layout_passes=False),
    scratch_types=(
        [pltpu.VMEM((N,), jnp.float32) for _ in range(2 * CPT)]
        + [
            pltpu.VMEM((N,), jnp.float32),
            pltpu.VMEM((2 * CHUNK,), jnp.int32),
            pltpu.SemaphoreType.DMA,
        ]
    ),
)
def _sc_kernel(xt_hbm, norm_hbm, pk_hbm, out_hbm, *refs):
    _sc_body(xt_hbm, norm_hbm, pk_hbm, out_hbm, refs)


def kernel(x, norm, edge_index):
    xt = x.reshape(N, NW, CPT).transpose(1, 2, 0)
    ei = edge_index.astype(jnp.int32)
    packed = (ei[1] << 14) | ei[0]
    out = _sc_kernel(xt, norm.reshape(N), packed)
    return out.transpose(2, 0, 1).reshape(N, D)
